# baseline (speedup 1.0000x reference)
import functools

import jax
import jax.numpy as jnp
from jax import lax
from jax.experimental import pallas as pl
from jax.experimental.pallas import tpu as pltpu
from jax.experimental.pallas import tpu_sc as plsc

N = 10000
E = 320000
D_IN, D_HID, D_OUT = 128, 64, 16

NC, NS = 2, 16
NW = NC * NS
EPW = E // NW
KB = 125
KN = EPW // KB
NBUF = 8
NG = KN // NBUF
DFL = 20
RPT = N // NS
NCH = RPT // KB


def _mesh():
    return plsc.VectorSubcoreMesh(
        core_axis_name="c", subcore_axis_name="s",
        num_cores=NC, num_subcores=NS)


_SC_PARAMS = pltpu.CompilerParams(use_tc_tiling_on_sc=False)


def _deg_body(dst_hbm, z1_hbm, out_hbm, acc, dst_v, ones_v, obuf, dsem):
    cid = lax.axis_index("c")
    sid = lax.axis_index("s")
    wid = sid * NC + cid

    @pl.when(sid < 10)
    def _init():
        pltpu.sync_copy(z1_hbm.at[pl.ds(sid * 1000, 1000)], obuf)
        pltpu.sync_copy(obuf, acc.at[pl.ds(sid * 1000, 1000)])

    for i in range(128 // 16):
        ones_v[pl.ds(i * 16, 16)] = jnp.ones((16,), jnp.float32)
    plsc.subcore_barrier()

    pltpu.sync_copy(dst_hbm.at[wid], dst_v)
    ones = ones_v.at[pl.ds(0, KB)]

    def group(g, c):
        for i in range(DFL):
            pltpu.async_copy(ones, acc.at[dst_v.at[g * DFL + i]],
                             dsem, add=True)
        for i in range(DFL):
            pltpu.make_async_copy(
                ones, acc.at[dst_v.at[g * DFL + i]], dsem).wait()
        return c
    lax.fori_loop(0, KN // DFL, group, 0)
    plsc.subcore_barrier()

    @pl.when(sid < 10)
    def _out():
        pltpu.sync_copy(acc.at[pl.ds(sid * 1000, 1000)], obuf)
        pltpu.sync_copy(obuf, out_hbm.at[pl.ds(cid * N + sid * 1000, 1000)])


_deg = functools.partial(
    pl.kernel,
    out_type=jax.ShapeDtypeStruct((NC * N,), jnp.float32),
    mesh=_mesh(),
    compiler_params=_SC_PARAMS,
    scratch_types=[
        pltpu.VMEM_SHARED((N,), jnp.float32),
        pltpu.VMEM((KN, KB), jnp.int32),
        pltpu.VMEM((128,), jnp.float32),
        pltpu.VMEM((1000,), jnp.float32),
        pltpu.SemaphoreType.DMA,
    ])(_deg_body)


def _agg_body(h_hbm, z_hbm, src_hbm, dst_hbm, out_hbm,
              acc, src_v, dst_v, *ring):
    rows = ring[:NBUF]
    gsem = ring[NBUF:2 * NBUF]
    ssem = ring[2 * NBUF:3 * NBUF]
    cid = lax.axis_index("c")
    sid = lax.axis_index("s")
    wid = sid * NC + cid

    for k in range(NCH):
        r0 = sid * RPT + k * KB
        pltpu.sync_copy(z_hbm.at[pl.ds(r0, KB)], rows[k % NBUF])
        pltpu.sync_copy(rows[k % NBUF], acc.at[pl.ds(r0, KB)])
    plsc.subcore_barrier()

    pltpu.sync_copy(src_hbm.at[wid], src_v)
    pltpu.sync_copy(dst_hbm.at[wid], dst_v)

    for b in range(NBUF):
        pltpu.async_copy(h_hbm.at[src_v.at[b]], rows[b], gsem[b])

    def group(g, c):
        descs = []
        for b in range(NBUF):
            j = g * NBUF + b
            pltpu.make_async_copy(
                h_hbm.at[src_v.at[j]], rows[b], gsem[b]).wait()
            descs.append(pltpu.async_copy(
                rows[b], acc.at[dst_v.at[j]], ssem[b], add=True))
        for b in range(NBUF):
            descs[b].wait()

            @pl.when(g < NG - 1)
            def _prefetch(b=b):
                jn = (g + 1) * NBUF + b
                pltpu.async_copy(h_hbm.at[src_v.at[jn]], rows[b], gsem[b])
        return c
    lax.fori_loop(0, NG, group, 0)
    plsc.subcore_barrier()

    for k in range(NCH):
        r0 = sid * RPT + k * KB
        pltpu.sync_copy(acc.at[pl.ds(r0, KB)], rows[k % NBUF])
        pltpu.sync_copy(rows[k % NBUF], out_hbm.at[cid, pl.ds(r0, KB)])


def _make_agg(d):
    return functools.partial(
        pl.kernel,
        out_type=jax.ShapeDtypeStruct((NC, N, d), jnp.float32),
        mesh=_mesh(),
        compiler_params=_SC_PARAMS,
        scratch_types=(
            [pltpu.VMEM_SHARED((N, d), jnp.float32),
             pltpu.VMEM((KN, KB), jnp.int32),
             pltpu.VMEM((KN, KB), jnp.int32)]
            + [pltpu.VMEM((KB, d), jnp.float32) for _ in range(NBUF)]
            + [pltpu.SemaphoreType.DMA for _ in range(2 * NBUF)]
        ))(_agg_body)


_agg64 = _make_agg(D_HID)
_agg16 = _make_agg(D_OUT)


BR = 400


def _tc1_body(x_ref, w_ref, deg_ref, h_ref, dinv_ref):
    dinv = lax.rsqrt(deg_ref[...] + 1.0)
    h = jnp.dot(x_ref[...], w_ref[...], preferred_element_type=jnp.float32)
    h_ref[...] = h * dinv
    dinv_ref[...] = dinv


_tc1 = pl.pallas_call(
    _tc1_body,
    grid=(N // BR,),
    in_specs=[pl.BlockSpec((BR, D_IN), lambda i: (i, 0)),
              pl.BlockSpec((D_IN, D_HID), lambda i: (0, 0)),
              pl.BlockSpec((BR, 1), lambda i: (i, 0))],
    out_specs=[pl.BlockSpec((BR, D_HID), lambda i: (i, 0)),
               pl.BlockSpec((BR, 1), lambda i: (i, 0))],
    out_shape=[jax.ShapeDtypeStruct((N, D_HID), jnp.float32),
               jax.ShapeDtypeStruct((N, 1), jnp.float32)])


def _tc_mid_body(p_ref, hp_ref, dinv_ref, b_ref, w_ref, out_ref):
    t = p_ref[0] + p_ref[1] + hp_ref[...]
    t = t * dinv_ref[...] + b_ref[...]
    t = jnp.maximum(t, 0.0)
    out_ref[...] = jnp.dot(
        t, w_ref[...], preferred_element_type=jnp.float32) * dinv_ref[...]


_tc_mid = pl.pallas_call(
    _tc_mid_body,
    grid=(N // BR,),
    in_specs=[pl.BlockSpec((NC, BR, D_HID), lambda i: (0, i, 0)),
              pl.BlockSpec((BR, D_HID), lambda i: (i, 0)),
              pl.BlockSpec((BR, 1), lambda i: (i, 0)),
              pl.BlockSpec((1, D_HID), lambda i: (0, 0)),
              pl.BlockSpec((D_HID, D_OUT), lambda i: (0, 0))],
    out_specs=pl.BlockSpec((BR, D_OUT), lambda i: (i, 0)),
    out_shape=jax.ShapeDtypeStruct((N, D_OUT), jnp.float32))


def _tc_out_body(p_ref, hp_ref, dinv_ref, b_ref, out_ref):
    t = (p_ref[0] + p_ref[1] + hp_ref[...]) * dinv_ref[...] + b_ref[...]
    m = jnp.max(t, axis=1, keepdims=True)
    e = jnp.exp(t - m)
    s = jnp.sum(e, axis=1, keepdims=True)
    out_ref[...] = (t - m) - jnp.log(s)


_tc_out = pl.pallas_call(
    _tc_out_body,
    grid=(N // BR,),
    in_specs=[pl.BlockSpec((NC, BR, D_OUT), lambda i: (0, i, 0)),
              pl.BlockSpec((BR, D_OUT), lambda i: (i, 0)),
              pl.BlockSpec((BR, 1), lambda i: (i, 0)),
              pl.BlockSpec((1, D_OUT), lambda i: (0, 0))],
    out_specs=pl.BlockSpec((BR, D_OUT), lambda i: (i, 0)),
    out_shape=jax.ShapeDtypeStruct((N, D_OUT), jnp.float32))


def kernel(x, edge_index, W1, b1, W2, b2):
    ei = edge_index.astype(jnp.int32)
    src3 = ei[0].reshape(NW, KN, KB)
    dst3 = ei[1].reshape(NW, KN, KB)
    z1 = jnp.zeros((N,), jnp.float32)
    z64 = jnp.zeros((N, D_HID), jnp.float32)
    z16 = jnp.zeros((N, D_OUT), jnp.float32)

    degp = _deg(dst3, z1).reshape(NC, N)
    deg = (degp[0] + degp[1]).reshape(N, 1)
    h1p, dinv = _tc1(x, W1, deg)
    p1 = _agg64(h1p, z64, src3, dst3)
    h2p = _tc_mid(p1, h1p, dinv, b1.reshape(1, D_HID), W2)
    p2 = _agg16(h2p, z16, src3, dst3)
    return _tc_out(p2, h2p, dinv, b2.reshape(1, D_OUT))


# speedup vs baseline: 1.0965x; 1.0965x over previous
import functools

import jax
import jax.numpy as jnp
from jax import lax
from jax.experimental import pallas as pl
from jax.experimental.pallas import tpu as pltpu
from jax.experimental.pallas import tpu_sc as plsc

N = 10000
E = 320000
D_IN, D_HID, D_OUT = 128, 64, 16

NC, NS = 2, 16
NW = NC * NS
EPW = E // NW
KB = 125
KN = EPW // KB
NBUF = 8
NG = KN // NBUF
DFL = 20
RPT = N // NS
NCH = RPT // KB


def _mesh():
    return plsc.VectorSubcoreMesh(
        core_axis_name="c", subcore_axis_name="s",
        num_cores=NC, num_subcores=NS)


_SC_PARAMS = pltpu.CompilerParams(use_tc_tiling_on_sc=False)


def _deg_body(dst_hbm, z1_hbm, out_hbm, acc, dst_v, ones_v, obuf, dsem):
    cid = lax.axis_index("c")
    sid = lax.axis_index("s")
    wid = sid * NC + cid

    @pl.when(sid < 10)
    def _init():
        pltpu.sync_copy(z1_hbm.at[pl.ds(sid * 1000, 1000)], obuf)
        pltpu.sync_copy(obuf, acc.at[pl.ds(sid * 1000, 1000)])

    for i in range(128 // 16):
        ones_v[pl.ds(i * 16, 16)] = jnp.ones((16,), jnp.float32)
    plsc.subcore_barrier()

    pltpu.sync_copy(dst_hbm.at[wid], dst_v)
    ones = ones_v.at[pl.ds(0, KB)]

    def group(g, c):
        for i in range(DFL):
            pltpu.async_copy(ones, acc.at[dst_v.at[g * DFL + i]],
                             dsem, add=True)
        for i in range(DFL):
            pltpu.make_async_copy(
                ones, acc.at[dst_v.at[g * DFL + i]], dsem).wait()
        return c
    lax.fori_loop(0, KN // DFL, group, 0)
    plsc.subcore_barrier()

    @pl.when(sid < 10)
    def _out():
        pltpu.sync_copy(acc.at[pl.ds(sid * 1000, 1000)], obuf)
        pltpu.sync_copy(obuf, out_hbm.at[pl.ds(cid * N + sid * 1000, 1000)])


_deg = functools.partial(
    pl.kernel,
    out_type=jax.ShapeDtypeStruct((NC * N,), jnp.float32),
    mesh=_mesh(),
    compiler_params=_SC_PARAMS,
    scratch_types=[
        pltpu.VMEM_SHARED((N,), jnp.float32),
        pltpu.VMEM((KN, KB), jnp.int32),
        pltpu.VMEM((128,), jnp.float32),
        pltpu.VMEM((1000,), jnp.float32),
        pltpu.SemaphoreType.DMA,
    ])(_deg_body)


def _agg_body(h_hbm, z_hbm, src_hbm, dst_hbm, out_hbm,
              acc, src_v, dst_v, *ring):
    rows = ring[:NBUF]
    gsem = ring[NBUF:2 * NBUF]
    ssem = ring[2 * NBUF:3 * NBUF]
    cid = lax.axis_index("c")
    sid = lax.axis_index("s")
    wid = sid * NC + cid

    for k in range(NCH):
        r0 = sid * RPT + k * KB
        pltpu.sync_copy(z_hbm.at[pl.ds(r0, KB)], rows[k % NBUF])
        pltpu.sync_copy(rows[k % NBUF], acc.at[pl.ds(r0, KB)])
    plsc.subcore_barrier()

    pltpu.sync_copy(src_hbm.at[wid], src_v)
    pltpu.sync_copy(dst_hbm.at[wid], dst_v)

    H = NBUF // 2
    for b in range(H):
        pltpu.async_copy(h_hbm.at[src_v.at[b]], rows[b], gsem[b])

    def group(g, c):
        for b in range(NBUF):
            j = g * NBUF + b
            pltpu.make_async_copy(
                h_hbm.at[src_v.at[j]], rows[b], gsem[b]).wait()
            pltpu.async_copy(rows[b], acc.at[dst_v.at[j]], ssem[b], add=True)
            h = (b + H) % NBUF
            jg = j + H

            @pl.when(jg < KN)
            def _prefetch(h=h, jg=jg):
                @pl.when(jg >= NBUF)
                def _wait_scatter():
                    pltpu.make_async_copy(
                        rows[h], acc.at[dst_v.at[jg - NBUF]], ssem[h]).wait()
                pltpu.async_copy(h_hbm.at[src_v.at[jg]], rows[h], gsem[h])
        return c
    lax.fori_loop(0, NG, group, 0)
    for b in range(NBUF):
        pltpu.make_async_copy(
            rows[b], acc.at[dst_v.at[KN - NBUF + b]], ssem[b]).wait()
    plsc.subcore_barrier()

    for k in range(NCH):
        r0 = sid * RPT + k * KB
        pltpu.sync_copy(acc.at[pl.ds(r0, KB)], rows[k % NBUF])
        pltpu.sync_copy(rows[k % NBUF], out_hbm.at[cid, pl.ds(r0, KB)])


def _make_agg(d):
    return functools.partial(
        pl.kernel,
        out_type=jax.ShapeDtypeStruct((NC, N, d), jnp.float32),
        mesh=_mesh(),
        compiler_params=_SC_PARAMS,
        scratch_types=(
            [pltpu.VMEM_SHARED((N, d), jnp.float32),
             pltpu.VMEM((KN, KB), jnp.int32),
             pltpu.VMEM((KN, KB), jnp.int32)]
            + [pltpu.VMEM((KB, d), jnp.float32) for _ in range(NBUF)]
            + [pltpu.SemaphoreType.DMA for _ in range(2 * NBUF)]
        ))(_agg_body)


_agg64 = _make_agg(D_HID)
_agg16 = _make_agg(D_OUT)


BR = 1000


def _tc1_body(x_ref, w_ref, deg_ref, h_ref, dinv_ref):
    dinv = lax.rsqrt(deg_ref[...] + 1.0)
    h = jnp.dot(x_ref[...], w_ref[...], preferred_element_type=jnp.float32)
    h_ref[...] = h * dinv
    dinv_ref[...] = dinv


_tc1 = pl.pallas_call(
    _tc1_body,
    grid=(N // BR,),
    in_specs=[pl.BlockSpec((BR, D_IN), lambda i: (i, 0)),
              pl.BlockSpec((D_IN, D_HID), lambda i: (0, 0)),
              pl.BlockSpec((BR, 1), lambda i: (i, 0))],
    out_specs=[pl.BlockSpec((BR, D_HID), lambda i: (i, 0)),
               pl.BlockSpec((BR, 1), lambda i: (i, 0))],
    out_shape=[jax.ShapeDtypeStruct((N, D_HID), jnp.float32),
               jax.ShapeDtypeStruct((N, 1), jnp.float32)])


def _tc_mid_body(p_ref, hp_ref, dinv_ref, b_ref, w_ref, out_ref):
    t = p_ref[0] + p_ref[1] + hp_ref[...]
    t = t * dinv_ref[...] + b_ref[...]
    t = jnp.maximum(t, 0.0)
    out_ref[...] = jnp.dot(
        t, w_ref[...], preferred_element_type=jnp.float32) * dinv_ref[...]


_tc_mid = pl.pallas_call(
    _tc_mid_body,
    grid=(N // BR,),
    in_specs=[pl.BlockSpec((NC, BR, D_HID), lambda i: (0, i, 0)),
              pl.BlockSpec((BR, D_HID), lambda i: (i, 0)),
              pl.BlockSpec((BR, 1), lambda i: (i, 0)),
              pl.BlockSpec((1, D_HID), lambda i: (0, 0)),
              pl.BlockSpec((D_HID, D_OUT), lambda i: (0, 0))],
    out_specs=pl.BlockSpec((BR, D_OUT), lambda i: (i, 0)),
    out_shape=jax.ShapeDtypeStruct((N, D_OUT), jnp.float32))


def _tc_out_body(p_ref, hp_ref, dinv_ref, b_ref, out_ref):
    t = (p_ref[0] + p_ref[1] + hp_ref[...]) * dinv_ref[...] + b_ref[...]
    m = jnp.max(t, axis=1, keepdims=True)
    e = jnp.exp(t - m)
    s = jnp.sum(e, axis=1, keepdims=True)
    out_ref[...] = (t - m) - jnp.log(s)


_tc_out = pl.pallas_call(
    _tc_out_body,
    grid=(N // BR,),
    in_specs=[pl.BlockSpec((NC, BR, D_OUT), lambda i: (0, i, 0)),
              pl.BlockSpec((BR, D_OUT), lambda i: (i, 0)),
              pl.BlockSpec((BR, 1), lambda i: (i, 0)),
              pl.BlockSpec((1, D_OUT), lambda i: (0, 0))],
    out_specs=pl.BlockSpec((BR, D_OUT), lambda i: (i, 0)),
    out_shape=jax.ShapeDtypeStruct((N, D_OUT), jnp.float32))


def kernel(x, edge_index, W1, b1, W2, b2):
    ei = edge_index.astype(jnp.int32)
    src3 = ei[0].reshape(NW, KN, KB)
    dst3 = ei[1].reshape(NW, KN, KB)
    z1 = jnp.zeros((N,), jnp.float32)
    z64 = jnp.zeros((N, D_HID), jnp.float32)
    z16 = jnp.zeros((N, D_OUT), jnp.float32)

    degp = _deg(dst3, z1).reshape(NC, N)
    deg = (degp[0] + degp[1]).reshape(N, 1)
    h1p, dinv = _tc1(x, W1, deg)
    p1 = _agg64(h1p, z64, src3, dst3)
    h2p = _tc_mid(p1, h1p, dinv, b1.reshape(1, D_HID), W2)
    p2 = _agg16(h2p, z16, src3, dst3)
    return _tc_out(p2, h2p, dinv, b2.reshape(1, D_OUT))
